# baseline (speedup 1.0000x reference)
import jax
import jax.numpy as jnp
from jax.experimental import pallas as pl

N = 65536
IN_FEATURES = 1024
OUT_FEATURES = 16
BLOCK_N = 4096


def _matmul_body(x_ref, wt_ref, b_ref, out_ref):
    out_ref[...] = (
        jnp.dot(x_ref[...], wt_ref[...], preferred_element_type=jnp.float32)
        + b_ref[...]
    )


def kernel(input, weight, bias):
    wt = weight.T
    b2 = bias.reshape(1, OUT_FEATURES)
    grid = (N // BLOCK_N,)
    return pl.pallas_call(
        _matmul_body,
        grid=grid,
        in_specs=[
            pl.BlockSpec((BLOCK_N, IN_FEATURES), lambda i: (i, 0)),
            pl.BlockSpec((IN_FEATURES, OUT_FEATURES), lambda i: (0, 0)),
            pl.BlockSpec((1, OUT_FEATURES), lambda i: (0, 0)),
        ],
        out_specs=pl.BlockSpec((BLOCK_N, OUT_FEATURES), lambda i: (i, 0)),
        out_shape=jax.ShapeDtypeStruct((N, OUT_FEATURES), jnp.float32),
    )(input, wt, b2)


# speedup vs baseline: 1.0017x; 1.0017x over previous
import jax
import jax.numpy as jnp
from jax.experimental import pallas as pl

N = 65536
IN_FEATURES = 1024
OUT_FEATURES = 16
BLOCK_N = 2048
N_STREAMS = 2
CHUNK_K = IN_FEATURES // N_STREAMS


def _matmul_body(*refs):
    x_refs = refs[:N_STREAMS]
    wt_ref, b_ref, out_ref = refs[N_STREAMS:]
    acc = b_ref[...]
    for s in range(N_STREAMS):
        acc = acc + jnp.dot(
            x_refs[s][...],
            wt_ref[pl.ds(s * CHUNK_K, CHUNK_K), :],
            preferred_element_type=jnp.float32,
        )
    out_ref[...] = acc


def kernel(input, weight, bias):
    wt = weight.T
    b2 = bias.reshape(1, OUT_FEATURES)
    grid = (N // BLOCK_N,)
    in_specs = [
        pl.BlockSpec((BLOCK_N, CHUNK_K), lambda i, s=s: (i, s))
        for s in range(N_STREAMS)
    ]
    in_specs.append(pl.BlockSpec((IN_FEATURES, OUT_FEATURES), lambda i: (0, 0)))
    in_specs.append(pl.BlockSpec((1, OUT_FEATURES), lambda i: (0, 0)))
    return pl.pallas_call(
        _matmul_body,
        grid=grid,
        in_specs=in_specs,
        out_specs=pl.BlockSpec((BLOCK_N, OUT_FEATURES), lambda i: (i, 0)),
        out_shape=jax.ShapeDtypeStruct((N, OUT_FEATURES), jnp.float32),
    )(*([input] * N_STREAMS), wt, b2)


# speedup vs baseline: 1.0107x; 1.0089x over previous
import jax
import jax.numpy as jnp
from jax.experimental import pallas as pl
from jax.experimental.pallas import tpu as pltpu

N = 65536
IN_FEATURES = 1024
OUT_FEATURES = 16
BLOCK_N = 1024
NBLK = N // BLOCK_N
NBUF = 4


def _body(x_hbm, wt_ref, b_ref, out_ref, bufs, sems):
    i = pl.program_id(0)

    def start_copy(b, slot):
        pltpu.make_async_copy(
            x_hbm.at[pl.ds(b * BLOCK_N, BLOCK_N), :],
            bufs.at[slot],
            sems.at[slot],
        ).start()

    @pl.when(i == 0)
    def _prologue():
        for j in range(NBUF - 1):
            start_copy(j, j)

    nxt = i + NBUF - 1

    @pl.when(nxt < NBLK)
    def _fetch_ahead():
        start_copy(nxt, nxt % NBUF)

    slot = i % NBUF
    pltpu.make_async_copy(
        x_hbm.at[pl.ds(i * BLOCK_N, BLOCK_N), :],
        bufs.at[slot],
        sems.at[slot],
    ).wait()
    out_ref[...] = (
        jnp.dot(bufs[slot], wt_ref[...], preferred_element_type=jnp.float32)
        + b_ref[...]
    )


def kernel(input, weight, bias):
    wt = weight.T
    b2 = bias.reshape(1, OUT_FEATURES)
    return pl.pallas_call(
        _body,
        grid=(NBLK,),
        in_specs=[
            pl.BlockSpec(memory_space=pl.ANY),
            pl.BlockSpec((IN_FEATURES, OUT_FEATURES), lambda i: (0, 0)),
            pl.BlockSpec((1, OUT_FEATURES), lambda i: (0, 0)),
        ],
        out_specs=pl.BlockSpec((BLOCK_N, OUT_FEATURES), lambda i: (i, 0)),
        out_shape=jax.ShapeDtypeStruct((N, OUT_FEATURES), jnp.float32),
        scratch_shapes=[
            pltpu.VMEM((NBUF, BLOCK_N, IN_FEATURES), jnp.float32),
            pltpu.SemaphoreType.DMA((NBUF,)),
        ],
    )(input, wt, b2)


# speedup vs baseline: 1.3473x; 1.3331x over previous
import jax
import jax.numpy as jnp
from jax.experimental import pallas as pl

N = 65536
IN_FEATURES = 1024
OUT_FEATURES = 16
BLOCK_N = 2048


def _matmul_body(x_ref, w_ref, b_ref, out_ref):
    acc = jax.lax.dot_general(
        x_ref[...],
        w_ref[...],
        dimension_numbers=(((1,), (1,)), ((), ())),
        preferred_element_type=jnp.float32,
    )
    out_ref[...] = acc.T + b_ref[...]


def kernel(input, weight, bias):
    b_col = bias.reshape(OUT_FEATURES, 1)
    out_t = pl.pallas_call(
        _matmul_body,
        grid=(N // BLOCK_N,),
        in_specs=[
            pl.BlockSpec((BLOCK_N, IN_FEATURES), lambda i: (i, 0)),
            pl.BlockSpec((OUT_FEATURES, IN_FEATURES), lambda i: (0, 0)),
            pl.BlockSpec((OUT_FEATURES, 1), lambda i: (0, 0)),
        ],
        out_specs=pl.BlockSpec((OUT_FEATURES, BLOCK_N), lambda i: (0, i)),
        out_shape=jax.ShapeDtypeStruct((OUT_FEATURES, N), jnp.float32),
    )(input, weight, b_col)
    return out_t.T


# speedup vs baseline: 1.3670x; 1.0146x over previous
import jax
import jax.numpy as jnp
from jax.experimental import pallas as pl

N = 65536
IN_FEATURES = 1024
OUT_FEATURES = 16
BLOCK_N = 2048


def _matmul_body(x_ref, w_ref, b_ref, out_ref):
    acc = jax.lax.dot_general(
        x_ref[...],
        w_ref[...],
        dimension_numbers=(((1,), (1,)), ((), ())),
        preferred_element_type=jnp.float32,
    )
    out_ref[...] = (acc + b_ref[...]).T


def kernel(input, weight, bias):
    b_row = bias.reshape(1, OUT_FEATURES)
    out_t = pl.pallas_call(
        _matmul_body,
        grid=(N // BLOCK_N,),
        in_specs=[
            pl.BlockSpec((BLOCK_N, IN_FEATURES), lambda i: (i, 0)),
            pl.BlockSpec((OUT_FEATURES, IN_FEATURES), lambda i: (0, 0)),
            pl.BlockSpec((1, OUT_FEATURES), lambda i: (0, 0)),
        ],
        out_specs=pl.BlockSpec((OUT_FEATURES, BLOCK_N), lambda i: (0, i)),
        out_shape=jax.ShapeDtypeStruct((OUT_FEATURES, N), jnp.float32),
    )(input, weight, b_row)
    return out_t.T
